# baseline (speedup 1.0000x reference)
import functools

import jax
import jax.numpy as jnp
from jax import lax
from jax.experimental import pallas as pl
from jax.experimental.pallas import tpu as pltpu
from jax.experimental.pallas import tpu_sc as plsc

_LANES = 16
_CHUNK = 128

_GDN = lax.GatherDimensionNumbers(offset_dims=(), collapsed_slice_dims=(0,),
                                  start_index_map=(0,))


def _splat(vec16, i):
    sel = jnp.full((_LANES, 1), i, jnp.int32)
    return lax.gather(vec16, sel, _GDN, slice_sizes=(1,),
                      mode=lax.GatherScatterMode.PROMISE_IN_BOUNDS)


def _rsqrt_sc(x):
    i = lax.bitcast_convert_type(x, jnp.int32)
    y = lax.bitcast_convert_type(
        jnp.full(x.shape, 0x5F3759DF, jnp.int32) - (i >> 1), jnp.float32)
    half = x * 0.5
    for _ in range(3):
        y = y * (1.5 - half * y * y)
    return y


def _deg_phase(dst_hbm, zeros1_hbm, deg_hbm, dst_v, alt_v, ones_v,
               dega_sh, c, s):
    nch = dst_hbm.shape[1]
    npad = zeros1_hbm.shape[0]
    rows_per_tile = npad // 16

    pltpu.sync_copy(zeros1_hbm.at[pl.ds(s * rows_per_tile, rows_per_tile)],
                    dega_sh.at[pl.ds(s * rows_per_tile, rows_per_tile)])
    for i in range(_CHUNK // _LANES):
        ones_v[pl.ds(i * _LANES, _LANES)] = jnp.ones((_LANES,), jnp.float32)
    pltpu.sync_copy(dst_hbm.at[2 * s], dst_v)
    pltpu.sync_copy(dst_hbm.at[2 * s + 1], alt_v)
    plsc.subcore_barrier()

    def body(j, carry):
        pltpu.sync_copy(ones_v, dega_sh.at[dst_v.at[j]], add=True)
        pltpu.sync_copy(ones_v, dega_sh.at[alt_v.at[j]], add=True)
        return carry

    lax.fori_loop(0, nch, body, 0, unroll=False)
    plsc.subcore_barrier()
    pltpu.sync_copy(dega_sh.at[pl.ds(s * rows_per_tile, rows_per_tile)],
                    deg_hbm.at[c, pl.ds(s * rows_per_tile, rows_per_tile)])


def _agg_phase(ytab_hbm, c, src_hbm, dst_hbm, zeros_hbm, out_hbm,
               src_v, dst_v, rows_v, sem, acc_sh, s):
    w = s * 2 + c
    nch = src_hbm.shape[1]
    npad = zeros_hbm.shape[0]
    rows_per_tile = npad // 16

    pltpu.sync_copy(zeros_hbm.at[pl.ds(s * rows_per_tile, rows_per_tile)],
                    acc_sh.at[pl.ds(s * rows_per_tile, rows_per_tile)])
    pltpu.sync_copy(src_hbm.at[w], src_v)
    pltpu.sync_copy(dst_hbm.at[w], dst_v)
    plsc.subcore_barrier()

    def body(j, carry):
        pltpu.async_copy(ytab_hbm.at[c].at[src_v.at[j]], rows_v, sem).wait()
        pltpu.sync_copy(rows_v, acc_sh.at[dst_v.at[j]], add=True)
        return carry

    lax.fori_loop(0, nch, body, 0, unroll=False)
    plsc.subcore_barrier()
    pltpu.sync_copy(acc_sh.at[pl.ds(s * rows_per_tile, rows_per_tile)],
                    out_hbm.at[c, pl.ds(s * rows_per_tile, rows_per_tile)])


def _sc_agg1(xw_hbm, src_hbm, dst_hbm, zeros1_hbm, zeros_hbm,
             out_hbm, deg_hbm, ytab_hbm,
             src_v, dst_v, rows_v, ones_v, dis_v, buf_v, sem, acc_sh,
             dega_sh):
    c = lax.axis_index("c")
    s = lax.axis_index("s")
    npad = zeros1_hbm.shape[0]
    rows_per_tile = npad // 16
    row0 = s * rows_per_tile

    _deg_phase(dst_hbm, zeros1_hbm, deg_hbm, dst_v, src_v, ones_v, dega_sh,
               c, s)

    pltpu.sync_copy(dega_sh.at[pl.ds(row0, rows_per_tile)], dis_v)
    for k in range(rows_per_tile // _LANES):
        sl = pl.ds(k * _LANES, _LANES)
        dis_v[sl] = _rsqrt_sc(dis_v[sl] + 1.0)

    pltpu.sync_copy(xw_hbm.at[pl.ds(row0, rows_per_tile)], buf_v)

    def ychunk(k, carry):
        dis16 = dis_v[pl.ds(k * _LANES, _LANES)]
        for i in range(_LANES):
            dsp = _splat(dis16, i)
            r = k * _LANES + i
            buf_v[r] = dsp * buf_v[r]
        return carry

    lax.fori_loop(0, rows_per_tile // _LANES, ychunk, 0, unroll=False)
    pltpu.sync_copy(buf_v, ytab_hbm.at[c, pl.ds(row0, rows_per_tile)])

    _agg_phase(ytab_hbm, c, src_hbm, dst_hbm, zeros_hbm, out_hbm,
               src_v, dst_v, rows_v, sem, acc_sh, s)


def _sc_agg2(p_hbm, xw_hbm, deg_hbm, b1_hbm, src_hbm, dst_hbm, zeros_hbm,
             out_hbm, ytab_hbm,
             src_v, dst_v, rows_v, dis_v, inv_v, buf_v, p0_v, p1_v, b1_v,
             sem, acc_sh):
    c = lax.axis_index("c")
    s = lax.axis_index("s")
    npad = deg_hbm.shape[1]
    rows_per_tile = npad // 16
    row0 = s * rows_per_tile

    pltpu.sync_copy(b1_hbm, b1_v)
    pltpu.sync_copy(deg_hbm.at[c, pl.ds(row0, rows_per_tile)], dis_v)
    for k in range(rows_per_tile // _LANES):
        sl = pl.ds(k * _LANES, _LANES)
        deg = dis_v[sl] + 1.0
        dis_v[sl] = _rsqrt_sc(deg)
        inv_v[sl] = 1.0 / deg

    pltpu.sync_copy(xw_hbm.at[pl.ds(row0, rows_per_tile)], buf_v)
    pltpu.sync_copy(p_hbm.at[0, pl.ds(row0, rows_per_tile)], p0_v)
    pltpu.sync_copy(p_hbm.at[1, pl.ds(row0, rows_per_tile)], p1_v)

    def ychunk(k, carry):
        dis16 = dis_v[pl.ds(k * _LANES, _LANES)]
        inv16 = inv_v[pl.ds(k * _LANES, _LANES)]
        b1r = b1_v[...]
        for i in range(_LANES):
            dsp = _splat(dis16, i)
            isp = _splat(inv16, i)
            r = k * _LANES + i
            pre = dsp * (p0_v[r] + p1_v[r]) + buf_v[r] * isp + b1r
            h = jnp.maximum(pre, 0.0)
            buf_v[r] = dsp * h
        return carry

    lax.fori_loop(0, rows_per_tile // _LANES, ychunk, 0, unroll=False)
    pltpu.sync_copy(buf_v, ytab_hbm.at[c, pl.ds(row0, rows_per_tile)])

    _agg_phase(ytab_hbm, c, src_hbm, dst_hbm, zeros_hbm, out_hbm,
               src_v, dst_v, rows_v, sem, acc_sh, s)


def _run_sc_agg1(xw, src_chunks, dst_chunks):
    mesh = plsc.VectorSubcoreMesh(core_axis_name="c", subcore_axis_name="s")
    nch = src_chunks.shape[1]
    np_rows = xw.shape[0]
    rpt = np_rows // 16
    zeros1 = jnp.zeros((np_rows,), jnp.float32)
    zeros = jnp.zeros((np_rows, _LANES), jnp.float32)
    kern = pl.kernel(
        _sc_agg1,
        mesh=mesh,
        compiler_params=pltpu.CompilerParams(use_tc_tiling_on_sc=False),
        out_type=(
            jax.ShapeDtypeStruct((2, np_rows, _LANES), jnp.float32),
            jax.ShapeDtypeStruct((2, np_rows), jnp.float32),
            jax.ShapeDtypeStruct((2, np_rows, _LANES), jnp.float32),
        ),
        scratch_types=[
            pltpu.VMEM((nch, _CHUNK), jnp.int32),
            pltpu.VMEM((nch, _CHUNK), jnp.int32),
            pltpu.VMEM((_CHUNK, _LANES), jnp.float32),
            pltpu.VMEM((_CHUNK,), jnp.float32),
            pltpu.VMEM((rpt,), jnp.float32),
            pltpu.VMEM((rpt, _LANES), jnp.float32),
            pltpu.SemaphoreType.DMA,
            pltpu.VMEM_SHARED((np_rows, _LANES), jnp.float32),
            pltpu.VMEM_SHARED((np_rows,), jnp.float32),
        ],
    )
    p, deg, _ = kern(xw, src_chunks, dst_chunks, zeros1, zeros)
    return p, deg


def _run_sc_agg2(p, xw, deg, b1, src_chunks, dst_chunks):
    mesh = plsc.VectorSubcoreMesh(core_axis_name="c", subcore_axis_name="s")
    nch = src_chunks.shape[1]
    np_rows = xw.shape[0]
    rpt = np_rows // 16
    zeros = jnp.zeros((np_rows, _LANES), jnp.float32)
    kern = pl.kernel(
        _sc_agg2,
        mesh=mesh,
        compiler_params=pltpu.CompilerParams(use_tc_tiling_on_sc=False),
        out_type=(
            jax.ShapeDtypeStruct((2, np_rows, _LANES), jnp.float32),
            jax.ShapeDtypeStruct((2, np_rows, _LANES), jnp.float32),
        ),
        scratch_types=[
            pltpu.VMEM((nch, _CHUNK), jnp.int32),
            pltpu.VMEM((nch, _CHUNK), jnp.int32),
            pltpu.VMEM((_CHUNK, _LANES), jnp.float32),
            pltpu.VMEM((rpt,), jnp.float32),
            pltpu.VMEM((rpt,), jnp.float32),
            pltpu.VMEM((rpt, _LANES), jnp.float32),
            pltpu.VMEM((rpt, _LANES), jnp.float32),
            pltpu.VMEM((rpt, _LANES), jnp.float32),
            pltpu.VMEM((_LANES,), jnp.float32),
            pltpu.SemaphoreType.DMA,
            pltpu.VMEM_SHARED((np_rows, _LANES), jnp.float32),
        ],
    )
    q, _ = kern(p, xw, deg, b1, src_chunks, dst_chunks, zeros)
    return q


def _tc_dense1(x_ref, w1_ref, o_xw):
    o_xw[...] = jnp.dot(x_ref[...], w1_ref[...],
                        preferred_element_type=jnp.float32)


def _tc_dense3(q_ref, p_ref, xw_ref, deg_ref, b1_ref, w2_ref, b2_ref, o_out):
    n_real = o_out.shape[1]
    deg = deg_ref[0] + 1.0
    deg = deg[:, None]
    dis = lax.rsqrt(deg)
    xw = xw_ref[...]
    h = jax.nn.relu(dis * (p_ref[0] + p_ref[1]) + xw / deg
                    + b1_ref[...][None, :])
    o = (q_ref[0] + q_ref[1]) * dis + h / deg
    logits_t = lax.dot_general(w2_ref[...], o, (((0,), (1,)), ((), ())),
                               preferred_element_type=jnp.float32)
    logits_t = logits_t + b2_ref[...][:, None]
    m = jnp.max(logits_t, axis=0, keepdims=True)
    lse = m + jnp.log(jnp.sum(jnp.exp(logits_t - m), axis=0, keepdims=True))
    o_out[...] = (logits_t - lse)[:, :n_real]


def kernel(x, edge_index, W1, b1, W2, b2):
    n, f_in = x.shape
    f_hid = W1.shape[1]
    f_out = W2.shape[1]
    e = edge_index.shape[1]

    np_rows = ((n // 256) + 1) * 256
    n_workers = 32
    nch = -(-e // (n_workers * _CHUNK))
    ep = n_workers * nch * _CHUNK
    n_spread = np_rows - n

    src = edge_index[0].astype(jnp.int32)
    dst = edge_index[1].astype(jnp.int32)
    pad_idx = n + jnp.arange(ep - e, dtype=jnp.int32) % n_spread
    src_chunks = jnp.concatenate([src, pad_idx]).reshape(n_workers, nch, _CHUNK)
    dst_chunks = jnp.concatenate([dst, pad_idx]).reshape(n_workers, nch, _CHUNK)

    x_pad = jnp.pad(x, ((0, np_rows - n), (0, 0)))

    xw = pl.pallas_call(
        _tc_dense1,
        out_shape=jax.ShapeDtypeStruct((np_rows, f_hid), jnp.float32),
    )(x_pad, W1)

    p, deg = _run_sc_agg1(xw, src_chunks, dst_chunks)

    q = _run_sc_agg2(p, xw, deg, b1, src_chunks, dst_chunks)

    out_t = pl.pallas_call(
        _tc_dense3,
        out_shape=jax.ShapeDtypeStruct((f_out, n), jnp.float32),
    )(q, p, xw, deg, b1, W2, b2)

    return out_t.T


# speedup vs baseline: 1.0682x; 1.0682x over previous
import functools

import jax
import jax.numpy as jnp
from jax import lax
from jax.experimental import pallas as pl
from jax.experimental.pallas import tpu as pltpu
from jax.experimental.pallas import tpu_sc as plsc

_LANES = 16
_CHUNK = 128

_GDN = lax.GatherDimensionNumbers(offset_dims=(), collapsed_slice_dims=(0,),
                                  start_index_map=(0,))


def _splat(vec16, i):
    sel = jnp.full((_LANES, 1), i, jnp.int32)
    return lax.gather(vec16, sel, _GDN, slice_sizes=(1,),
                      mode=lax.GatherScatterMode.PROMISE_IN_BOUNDS)


def _rsqrt_sc(x):
    i = lax.bitcast_convert_type(x, jnp.int32)
    y = lax.bitcast_convert_type(
        jnp.full(x.shape, 0x5F3759DF, jnp.int32) - (i >> 1), jnp.float32)
    half = x * 0.5
    for _ in range(3):
        y = y * (1.5 - half * y * y)
    return y


def _sc_degree(dst_hbm, zeros_hbm, out_hbm, dst_v, ones_v, acc_sh):
    c = lax.axis_index("c")
    s = lax.axis_index("s")
    w = s * 2 + c
    nch = dst_hbm.shape[1]
    npad = zeros_hbm.shape[0]
    rows_per_tile = npad // 16

    pltpu.sync_copy(zeros_hbm.at[pl.ds(s * rows_per_tile, rows_per_tile)],
                    acc_sh.at[pl.ds(s * rows_per_tile, rows_per_tile)])
    for i in range(_CHUNK // _LANES):
        ones_v[pl.ds(i * _LANES, _LANES)] = jnp.ones((_LANES,), jnp.float32)
    pltpu.sync_copy(dst_hbm.at[w], dst_v)
    plsc.subcore_barrier()

    def body(j, carry):
        pltpu.sync_copy(ones_v, acc_sh.at[dst_v.at[j]], add=True)
        return carry

    lax.fori_loop(0, nch, body, 0, unroll=False)
    plsc.subcore_barrier()
    pltpu.sync_copy(acc_sh.at[pl.ds(s * rows_per_tile, rows_per_tile)],
                    out_hbm.at[c, pl.ds(s * rows_per_tile, rows_per_tile)])


def _dis_from_parts(dp_hbm, row0, rows_per_tile, dis_v, inv_v, want_inv):
    pltpu.sync_copy(dp_hbm.at[0, pl.ds(row0, rows_per_tile)], dis_v)
    pltpu.sync_copy(dp_hbm.at[1, pl.ds(row0, rows_per_tile)], inv_v)
    for k in range(rows_per_tile // _LANES):
        sl = pl.ds(k * _LANES, _LANES)
        deg = dis_v[sl] + inv_v[sl] + 1.0
        dis_v[sl] = _rsqrt_sc(deg)
        if want_inv:
            inv_v[sl] = 1.0 / deg


def _agg_phase(ytab_hbm, c, src_hbm, dst_hbm, zeros_hbm, out_hbm,
               src_v, dst_v, rows_v, sem, acc_sh, s):
    w = s * 2 + c
    nch = src_hbm.shape[1]
    npad = zeros_hbm.shape[0]
    rows_per_tile = npad // 16

    pltpu.sync_copy(zeros_hbm.at[pl.ds(s * rows_per_tile, rows_per_tile)],
                    acc_sh.at[pl.ds(s * rows_per_tile, rows_per_tile)])
    pltpu.sync_copy(src_hbm.at[w], src_v)
    pltpu.sync_copy(dst_hbm.at[w], dst_v)
    plsc.subcore_barrier()

    def body(j, carry):
        pltpu.async_copy(ytab_hbm.at[c].at[src_v.at[j]], rows_v, sem).wait()
        pltpu.sync_copy(rows_v, acc_sh.at[dst_v.at[j]], add=True)
        return carry

    lax.fori_loop(0, nch, body, 0, unroll=False)
    plsc.subcore_barrier()
    pltpu.sync_copy(acc_sh.at[pl.ds(s * rows_per_tile, rows_per_tile)],
                    out_hbm.at[c, pl.ds(s * rows_per_tile, rows_per_tile)])


def _sc_agg1(xw_hbm, dp_hbm, src_hbm, dst_hbm, zeros_hbm,
             out_hbm, ytab_hbm,
             src_v, dst_v, rows_v, dis_v, inv_v, buf_v, sem, acc_sh):
    c = lax.axis_index("c")
    s = lax.axis_index("s")
    npad = dp_hbm.shape[1]
    rows_per_tile = npad // 16
    row0 = s * rows_per_tile

    _dis_from_parts(dp_hbm, row0, rows_per_tile, dis_v, inv_v, want_inv=False)
    pltpu.sync_copy(xw_hbm.at[pl.ds(row0, rows_per_tile)], buf_v)

    def ychunk(k, carry):
        dis16 = dis_v[pl.ds(k * _LANES, _LANES)]
        for i in range(_LANES):
            dsp = _splat(dis16, i)
            r = k * _LANES + i
            buf_v[r] = dsp * buf_v[r]
        return carry

    lax.fori_loop(0, rows_per_tile // _LANES, ychunk, 0, unroll=False)
    pltpu.sync_copy(buf_v, ytab_hbm.at[c, pl.ds(row0, rows_per_tile)])

    _agg_phase(ytab_hbm, c, src_hbm, dst_hbm, zeros_hbm, out_hbm,
               src_v, dst_v, rows_v, sem, acc_sh, s)


def _sc_agg2(p_hbm, xw_hbm, dp_hbm, b1_hbm, src_hbm, dst_hbm, zeros_hbm,
             out_hbm, ytab_hbm,
             src_v, dst_v, rows_v, dis_v, inv_v, buf_v, p0_v, p1_v, b1_v,
             sem, acc_sh):
    c = lax.axis_index("c")
    s = lax.axis_index("s")
    npad = dp_hbm.shape[1]
    rows_per_tile = npad // 16
    row0 = s * rows_per_tile

    pltpu.sync_copy(b1_hbm, b1_v)
    _dis_from_parts(dp_hbm, row0, rows_per_tile, dis_v, inv_v, want_inv=True)
    pltpu.sync_copy(xw_hbm.at[pl.ds(row0, rows_per_tile)], buf_v)
    pltpu.sync_copy(p_hbm.at[0, pl.ds(row0, rows_per_tile)], p0_v)
    pltpu.sync_copy(p_hbm.at[1, pl.ds(row0, rows_per_tile)], p1_v)

    def ychunk(k, carry):
        dis16 = dis_v[pl.ds(k * _LANES, _LANES)]
        inv16 = inv_v[pl.ds(k * _LANES, _LANES)]
        b1r = b1_v[...]
        for i in range(_LANES):
            dsp = _splat(dis16, i)
            isp = _splat(inv16, i)
            r = k * _LANES + i
            pre = dsp * (p0_v[r] + p1_v[r]) + buf_v[r] * isp + b1r
            h = jnp.maximum(pre, 0.0)
            buf_v[r] = dsp * h
        return carry

    lax.fori_loop(0, rows_per_tile // _LANES, ychunk, 0, unroll=False)
    pltpu.sync_copy(buf_v, ytab_hbm.at[c, pl.ds(row0, rows_per_tile)])

    _agg_phase(ytab_hbm, c, src_hbm, dst_hbm, zeros_hbm, out_hbm,
               src_v, dst_v, rows_v, sem, acc_sh, s)


def _run_sc_degree(dst_chunks, np_rows):
    mesh = plsc.VectorSubcoreMesh(core_axis_name="c", subcore_axis_name="s")
    nch = dst_chunks.shape[1]
    zeros = jnp.zeros((np_rows,), jnp.float32)
    kern = pl.kernel(
        _sc_degree,
        mesh=mesh,
        compiler_params=pltpu.CompilerParams(use_tc_tiling_on_sc=False),
        out_type=jax.ShapeDtypeStruct((2, np_rows), jnp.float32),
        scratch_types=[
            pltpu.VMEM((nch, _CHUNK), jnp.int32),
            pltpu.VMEM((_CHUNK,), jnp.float32),
            pltpu.VMEM_SHARED((np_rows,), jnp.float32),
        ],
    )
    return kern(dst_chunks, zeros)


def _run_sc_agg1(xw, dp, src_chunks, dst_chunks):
    mesh = plsc.VectorSubcoreMesh(core_axis_name="c", subcore_axis_name="s")
    nch = src_chunks.shape[1]
    np_rows = xw.shape[0]
    rpt = np_rows // 16
    zeros = jnp.zeros((np_rows, _LANES), jnp.float32)
    kern = pl.kernel(
        _sc_agg1,
        mesh=mesh,
        compiler_params=pltpu.CompilerParams(use_tc_tiling_on_sc=False),
        out_type=(
            jax.ShapeDtypeStruct((2, np_rows, _LANES), jnp.float32),
            jax.ShapeDtypeStruct((2, np_rows, _LANES), jnp.float32),
        ),
        scratch_types=[
            pltpu.VMEM((nch, _CHUNK), jnp.int32),
            pltpu.VMEM((nch, _CHUNK), jnp.int32),
            pltpu.VMEM((_CHUNK, _LANES), jnp.float32),
            pltpu.VMEM((rpt,), jnp.float32),
            pltpu.VMEM((rpt,), jnp.float32),
            pltpu.VMEM((rpt, _LANES), jnp.float32),
            pltpu.SemaphoreType.DMA,
            pltpu.VMEM_SHARED((np_rows, _LANES), jnp.float32),
        ],
    )
    p, _ = kern(xw, dp, src_chunks, dst_chunks, zeros)
    return p


def _run_sc_agg2(p, xw, dp, b1, src_chunks, dst_chunks):
    mesh = plsc.VectorSubcoreMesh(core_axis_name="c", subcore_axis_name="s")
    nch = src_chunks.shape[1]
    np_rows = xw.shape[0]
    rpt = np_rows // 16
    zeros = jnp.zeros((np_rows, _LANES), jnp.float32)
    kern = pl.kernel(
        _sc_agg2,
        mesh=mesh,
        compiler_params=pltpu.CompilerParams(use_tc_tiling_on_sc=False),
        out_type=(
            jax.ShapeDtypeStruct((2, np_rows, _LANES), jnp.float32),
            jax.ShapeDtypeStruct((2, np_rows, _LANES), jnp.float32),
        ),
        scratch_types=[
            pltpu.VMEM((nch, _CHUNK), jnp.int32),
            pltpu.VMEM((nch, _CHUNK), jnp.int32),
            pltpu.VMEM((_CHUNK, _LANES), jnp.float32),
            pltpu.VMEM((rpt,), jnp.float32),
            pltpu.VMEM((rpt,), jnp.float32),
            pltpu.VMEM((rpt, _LANES), jnp.float32),
            pltpu.VMEM((rpt, _LANES), jnp.float32),
            pltpu.VMEM((rpt, _LANES), jnp.float32),
            pltpu.VMEM((_LANES,), jnp.float32),
            pltpu.SemaphoreType.DMA,
            pltpu.VMEM_SHARED((np_rows, _LANES), jnp.float32),
        ],
    )
    q, _ = kern(p, xw, dp, b1, src_chunks, dst_chunks, zeros)
    return q


def _tc_dense1(x_ref, w1_ref, o_xw):
    o_xw[...] = jnp.dot(x_ref[...], w1_ref[...],
                        preferred_element_type=jnp.float32)


def _tc_dense3(q_ref, p_ref, xw_ref, dp_ref, b1_ref, w2_ref, b2_ref, o_out):
    n_real = o_out.shape[1]
    deg = dp_ref[0] + dp_ref[1] + 1.0
    deg = deg[:, None]
    dis = lax.rsqrt(deg)
    xw = xw_ref[...]
    h = jax.nn.relu(dis * (p_ref[0] + p_ref[1]) + xw / deg
                    + b1_ref[...][None, :])
    o = (q_ref[0] + q_ref[1]) * dis + h / deg
    logits_t = lax.dot_general(w2_ref[...], o, (((0,), (1,)), ((), ())),
                               preferred_element_type=jnp.float32)
    logits_t = logits_t + b2_ref[...][:, None]
    m = jnp.max(logits_t, axis=0, keepdims=True)
    lse = m + jnp.log(jnp.sum(jnp.exp(logits_t - m), axis=0, keepdims=True))
    o_out[...] = (logits_t - lse)[:, :n_real]


def kernel(x, edge_index, W1, b1, W2, b2):
    n, f_in = x.shape
    f_hid = W1.shape[1]
    f_out = W2.shape[1]
    e = edge_index.shape[1]

    np_rows = ((n // 256) + 1) * 256
    n_workers = 32
    nch = -(-e // (n_workers * _CHUNK))
    ep = n_workers * nch * _CHUNK
    n_spread = np_rows - n

    src = edge_index[0].astype(jnp.int32)
    dst = edge_index[1].astype(jnp.int32)
    pad_idx = n + jnp.arange(ep - e, dtype=jnp.int32) % n_spread
    src_chunks = jnp.concatenate([src, pad_idx]).reshape(n_workers, nch, _CHUNK)
    dst_chunks = jnp.concatenate([dst, pad_idx]).reshape(n_workers, nch, _CHUNK)

    x_pad = jnp.pad(x, ((0, np_rows - n), (0, 0)))

    dp = _run_sc_degree(dst_chunks, np_rows)
    xw = pl.pallas_call(
        _tc_dense1,
        out_shape=jax.ShapeDtypeStruct((np_rows, f_hid), jnp.float32),
    )(x_pad, W1)

    p = _run_sc_agg1(xw, dp, src_chunks, dst_chunks)

    q = _run_sc_agg2(p, xw, dp, b1, src_chunks, dst_chunks)

    out_t = pl.pallas_call(
        _tc_dense3,
        out_shape=jax.ShapeDtypeStruct((f_out, n), jnp.float32),
    )(q, p, xw, dp, b1, W2, b2)

    return out_t.T


# speedup vs baseline: 1.0904x; 1.0208x over previous
import functools

import jax
import jax.numpy as jnp
from jax import lax
from jax.experimental import pallas as pl
from jax.experimental.pallas import tpu as pltpu
from jax.experimental.pallas import tpu_sc as plsc

_LANES = 16
_CHUNK = 128

_GDN = lax.GatherDimensionNumbers(offset_dims=(), collapsed_slice_dims=(0,),
                                  start_index_map=(0,))


def _splat(vec16, i):
    sel = jnp.full((_LANES, 1), i, jnp.int32)
    return lax.gather(vec16, sel, _GDN, slice_sizes=(1,),
                      mode=lax.GatherScatterMode.PROMISE_IN_BOUNDS)


def _rsqrt_sc(x):
    i = lax.bitcast_convert_type(x, jnp.int32)
    y = lax.bitcast_convert_type(
        jnp.full(x.shape, 0x5F3759DF, jnp.int32) - (i >> 1), jnp.float32)
    half = x * 0.5
    for _ in range(3):
        y = y * (1.5 - half * y * y)
    return y


def _sc_degree(dst_hbm, zeros_hbm, out_hbm, dst_v, ones_v, acc_sh):
    c = lax.axis_index("c")
    s = lax.axis_index("s")
    w = s * 2 + c
    nch = dst_hbm.shape[1]
    npad = zeros_hbm.shape[0]
    rows_per_tile = npad // 16

    pltpu.sync_copy(zeros_hbm.at[pl.ds(s * rows_per_tile, rows_per_tile)],
                    acc_sh.at[pl.ds(s * rows_per_tile, rows_per_tile)])
    for i in range(_CHUNK // _LANES):
        ones_v[pl.ds(i * _LANES, _LANES)] = jnp.ones((_LANES,), jnp.float32)
    pltpu.sync_copy(dst_hbm.at[w], dst_v)
    plsc.subcore_barrier()

    def body(j, carry):
        pltpu.sync_copy(ones_v, acc_sh.at[dst_v.at[j]], add=True)
        return carry

    lax.fori_loop(0, nch, body, 0, unroll=False)
    plsc.subcore_barrier()
    pltpu.sync_copy(acc_sh.at[pl.ds(s * rows_per_tile, rows_per_tile)],
                    out_hbm.at[c, pl.ds(s * rows_per_tile, rows_per_tile)])


def _dis_from_parts(dp_hbm, row0, rows_per_tile, dis_v, inv_v, want_inv):
    pltpu.sync_copy(dp_hbm.at[0, pl.ds(row0, rows_per_tile)], dis_v)
    pltpu.sync_copy(dp_hbm.at[1, pl.ds(row0, rows_per_tile)], inv_v)
    for k in range(rows_per_tile // _LANES):
        sl = pl.ds(k * _LANES, _LANES)
        deg = dis_v[sl] + inv_v[sl] + 1.0
        dis_v[sl] = _rsqrt_sc(deg)
        if want_inv:
            inv_v[sl] = 1.0 / deg


def _stage_agg_inputs(c, s, src_hbm, dst_hbm, zeros_hbm,
                      src_v, dst_v, acc_sh, sem):
    w = s * 2 + c
    npad = zeros_hbm.shape[0]
    rows_per_tile = npad // 16
    return (
        pltpu.async_copy(
            zeros_hbm.at[pl.ds(s * rows_per_tile, rows_per_tile)],
            acc_sh.at[pl.ds(s * rows_per_tile, rows_per_tile)], sem),
        pltpu.async_copy(src_hbm.at[w], src_v, sem),
        pltpu.async_copy(dst_hbm.at[w], dst_v, sem),
    )


def _agg_phase(ytab_hbm, c, src_hbm, dst_hbm, zeros_hbm, out_hbm,
               src_v, dst_v, rows_v, sem, acc_sh, s):
    nch = src_hbm.shape[1]
    npad = zeros_hbm.shape[0]
    rows_per_tile = npad // 16

    plsc.subcore_barrier()

    def body(j, carry):
        pltpu.async_copy(ytab_hbm.at[c].at[src_v.at[j]], rows_v, sem).wait()
        pltpu.sync_copy(rows_v, acc_sh.at[dst_v.at[j]], add=True)
        return carry

    lax.fori_loop(0, nch, body, 0, unroll=False)
    plsc.subcore_barrier()
    pltpu.sync_copy(acc_sh.at[pl.ds(s * rows_per_tile, rows_per_tile)],
                    out_hbm.at[c, pl.ds(s * rows_per_tile, rows_per_tile)])


def _sc_agg1(xw_hbm, dp_hbm, src_hbm, dst_hbm, zeros_hbm,
             out_hbm, ytab_hbm,
             src_v, dst_v, rows_v, dis_v, inv_v, buf_v, sem, acc_sh):
    c = lax.axis_index("c")
    s = lax.axis_index("s")
    npad = dp_hbm.shape[1]
    rows_per_tile = npad // 16
    row0 = s * rows_per_tile

    staged = _stage_agg_inputs(c, s, src_hbm, dst_hbm, zeros_hbm,
                               src_v, dst_v, acc_sh, sem)
    _dis_from_parts(dp_hbm, row0, rows_per_tile, dis_v, inv_v, want_inv=False)
    pltpu.sync_copy(xw_hbm.at[pl.ds(row0, rows_per_tile)], buf_v)

    def ychunk(k, carry):
        dis16 = dis_v[pl.ds(k * _LANES, _LANES)]
        for i in range(_LANES):
            dsp = _splat(dis16, i)
            r = k * _LANES + i
            buf_v[r] = dsp * buf_v[r]
        return carry

    lax.fori_loop(0, rows_per_tile // _LANES, ychunk, 0, unroll=False)
    pltpu.sync_copy(buf_v, ytab_hbm.at[c, pl.ds(row0, rows_per_tile)])
    for cp in staged:
        cp.wait()

    _agg_phase(ytab_hbm, c, src_hbm, dst_hbm, zeros_hbm, out_hbm,
               src_v, dst_v, rows_v, sem, acc_sh, s)


def _sc_agg2(p_hbm, xw_hbm, dp_hbm, b1_hbm, src_hbm, dst_hbm, zeros_hbm,
             out_hbm, ytab_hbm,
             src_v, dst_v, rows_v, dis_v, inv_v, buf_v, p0_v, p1_v, b1_v,
             sem, acc_sh):
    c = lax.axis_index("c")
    s = lax.axis_index("s")
    npad = dp_hbm.shape[1]
    rows_per_tile = npad // 16
    row0 = s * rows_per_tile

    staged = _stage_agg_inputs(c, s, src_hbm, dst_hbm, zeros_hbm,
                               src_v, dst_v, acc_sh, sem)
    pltpu.sync_copy(b1_hbm, b1_v)
    _dis_from_parts(dp_hbm, row0, rows_per_tile, dis_v, inv_v, want_inv=True)
    pltpu.sync_copy(xw_hbm.at[pl.ds(row0, rows_per_tile)], buf_v)
    pltpu.sync_copy(p_hbm.at[0, pl.ds(row0, rows_per_tile)], p0_v)
    pltpu.sync_copy(p_hbm.at[1, pl.ds(row0, rows_per_tile)], p1_v)

    def ychunk(k, carry):
        dis16 = dis_v[pl.ds(k * _LANES, _LANES)]
        inv16 = inv_v[pl.ds(k * _LANES, _LANES)]
        b1r = b1_v[...]
        for i in range(_LANES):
            dsp = _splat(dis16, i)
            isp = _splat(inv16, i)
            r = k * _LANES + i
            pre = dsp * (p0_v[r] + p1_v[r]) + buf_v[r] * isp + b1r
            h = jnp.maximum(pre, 0.0)
            buf_v[r] = dsp * h
        return carry

    lax.fori_loop(0, rows_per_tile // _LANES, ychunk, 0, unroll=False)
    pltpu.sync_copy(buf_v, ytab_hbm.at[c, pl.ds(row0, rows_per_tile)])
    for cp in staged:
        cp.wait()

    _agg_phase(ytab_hbm, c, src_hbm, dst_hbm, zeros_hbm, out_hbm,
               src_v, dst_v, rows_v, sem, acc_sh, s)


def _run_sc_degree(dst_chunks, np_rows):
    mesh = plsc.VectorSubcoreMesh(core_axis_name="c", subcore_axis_name="s")
    nch = dst_chunks.shape[1]
    zeros = jnp.zeros((np_rows,), jnp.float32)
    kern = pl.kernel(
        _sc_degree,
        mesh=mesh,
        compiler_params=pltpu.CompilerParams(use_tc_tiling_on_sc=False),
        out_type=jax.ShapeDtypeStruct((2, np_rows), jnp.float32),
        scratch_types=[
            pltpu.VMEM((nch, _CHUNK), jnp.int32),
            pltpu.VMEM((_CHUNK,), jnp.float32),
            pltpu.VMEM_SHARED((np_rows,), jnp.float32),
        ],
    )
    return kern(dst_chunks, zeros)


def _run_sc_agg1(xw, dp, src_chunks, dst_chunks):
    mesh = plsc.VectorSubcoreMesh(core_axis_name="c", subcore_axis_name="s")
    nch = src_chunks.shape[1]
    np_rows = xw.shape[0]
    rpt = np_rows // 16
    zeros = jnp.zeros((np_rows, _LANES), jnp.float32)
    kern = pl.kernel(
        _sc_agg1,
        mesh=mesh,
        compiler_params=pltpu.CompilerParams(use_tc_tiling_on_sc=False),
        out_type=(
            jax.ShapeDtypeStruct((2, np_rows, _LANES), jnp.float32),
            jax.ShapeDtypeStruct((2, np_rows, _LANES), jnp.float32),
        ),
        scratch_types=[
            pltpu.VMEM((nch, _CHUNK), jnp.int32),
            pltpu.VMEM((nch, _CHUNK), jnp.int32),
            pltpu.VMEM((_CHUNK, _LANES), jnp.float32),
            pltpu.VMEM((rpt,), jnp.float32),
            pltpu.VMEM((rpt,), jnp.float32),
            pltpu.VMEM((rpt, _LANES), jnp.float32),
            pltpu.SemaphoreType.DMA,
            pltpu.VMEM_SHARED((np_rows, _LANES), jnp.float32),
        ],
    )
    p, _ = kern(xw, dp, src_chunks, dst_chunks, zeros)
    return p


def _run_sc_agg2(p, xw, dp, b1, src_chunks, dst_chunks):
    mesh = plsc.VectorSubcoreMesh(core_axis_name="c", subcore_axis_name="s")
    nch = src_chunks.shape[1]
    np_rows = xw.shape[0]
    rpt = np_rows // 16
    zeros = jnp.zeros((np_rows, _LANES), jnp.float32)
    kern = pl.kernel(
        _sc_agg2,
        mesh=mesh,
        compiler_params=pltpu.CompilerParams(use_tc_tiling_on_sc=False),
        out_type=(
            jax.ShapeDtypeStruct((2, np_rows, _LANES), jnp.float32),
            jax.ShapeDtypeStruct((2, np_rows, _LANES), jnp.float32),
        ),
        scratch_types=[
            pltpu.VMEM((nch, _CHUNK), jnp.int32),
            pltpu.VMEM((nch, _CHUNK), jnp.int32),
            pltpu.VMEM((_CHUNK, _LANES), jnp.float32),
            pltpu.VMEM((rpt,), jnp.float32),
            pltpu.VMEM((rpt,), jnp.float32),
            pltpu.VMEM((rpt, _LANES), jnp.float32),
            pltpu.VMEM((rpt, _LANES), jnp.float32),
            pltpu.VMEM((rpt, _LANES), jnp.float32),
            pltpu.VMEM((_LANES,), jnp.float32),
            pltpu.SemaphoreType.DMA,
            pltpu.VMEM_SHARED((np_rows, _LANES), jnp.float32),
        ],
    )
    q, _ = kern(p, xw, dp, b1, src_chunks, dst_chunks, zeros)
    return q


def _tc_dense1(x_ref, w1_ref, o_xw):
    o_xw[...] = jnp.dot(x_ref[...], w1_ref[...],
                        preferred_element_type=jnp.float32)


def _tc_dense3(q_ref, p_ref, xw_ref, dp_ref, b1_ref, w2_ref, b2_ref, o_out):
    n_real = o_out.shape[1]
    deg = dp_ref[0] + dp_ref[1] + 1.0
    deg = deg[:, None]
    dis = lax.rsqrt(deg)
    xw = xw_ref[...]
    h = jax.nn.relu(dis * (p_ref[0] + p_ref[1]) + xw / deg
                    + b1_ref[...][None, :])
    o = (q_ref[0] + q_ref[1]) * dis + h / deg
    logits_t = lax.dot_general(w2_ref[...], o, (((0,), (1,)), ((), ())),
                               preferred_element_type=jnp.float32)
    logits_t = logits_t + b2_ref[...][:, None]
    m = jnp.max(logits_t, axis=0, keepdims=True)
    lse = m + jnp.log(jnp.sum(jnp.exp(logits_t - m), axis=0, keepdims=True))
    o_out[...] = (logits_t - lse)[:, :n_real]


def kernel(x, edge_index, W1, b1, W2, b2):
    n, f_in = x.shape
    f_hid = W1.shape[1]
    f_out = W2.shape[1]
    e = edge_index.shape[1]

    np_rows = ((n // 256) + 1) * 256
    n_workers = 32
    nch = -(-e // (n_workers * _CHUNK))
    ep = n_workers * nch * _CHUNK
    n_spread = np_rows - n

    src = edge_index[0].astype(jnp.int32)
    dst = edge_index[1].astype(jnp.int32)
    pad_idx = n + jnp.arange(ep - e, dtype=jnp.int32) % n_spread
    src_chunks = jnp.concatenate([src, pad_idx]).reshape(n_workers, nch, _CHUNK)
    dst_chunks = jnp.concatenate([dst, pad_idx]).reshape(n_workers, nch, _CHUNK)

    x_pad = jnp.pad(x, ((0, np_rows - n), (0, 0)))

    dp = _run_sc_degree(dst_chunks, np_rows)
    xw = pl.pallas_call(
        _tc_dense1,
        out_shape=jax.ShapeDtypeStruct((np_rows, f_hid), jnp.float32),
    )(x_pad, W1)

    p = _run_sc_agg1(xw, dp, src_chunks, dst_chunks)

    q = _run_sc_agg2(p, xw, dp, b1, src_chunks, dst_chunks)

    out_t = pl.pallas_call(
        _tc_dense3,
        out_shape=jax.ShapeDtypeStruct((f_out, n), jnp.float32),
    )(q, p, xw, dp, b1, W2, b2)

    return out_t.T


# speedup vs baseline: 1.0905x; 1.0001x over previous
import functools

import jax
import jax.numpy as jnp
from jax import lax
from jax.experimental import pallas as pl
from jax.experimental.pallas import tpu as pltpu
from jax.experimental.pallas import tpu_sc as plsc

_LANES = 16
_CHUNK = 128

_GDN = lax.GatherDimensionNumbers(offset_dims=(), collapsed_slice_dims=(0,),
                                  start_index_map=(0,))


def _splat(vec16, i):
    sel = jnp.full((_LANES, 1), i, jnp.int32)
    return lax.gather(vec16, sel, _GDN, slice_sizes=(1,),
                      mode=lax.GatherScatterMode.PROMISE_IN_BOUNDS)


def _rsqrt_sc(x):
    i = lax.bitcast_convert_type(x, jnp.int32)
    y = lax.bitcast_convert_type(
        jnp.full(x.shape, 0x5F3759DF, jnp.int32) - (i >> 1), jnp.float32)
    half = x * 0.5
    for _ in range(3):
        y = y * (1.5 - half * y * y)
    return y


def _sc_degree(dst_hbm, zeros_hbm, out_hbm, dst_v, ones_v, acc_sh):
    c = lax.axis_index("c")
    s = lax.axis_index("s")
    w = s * 2 + c
    nch = dst_hbm.shape[1]
    npad = zeros_hbm.shape[0]
    rows_per_tile = npad // 16

    pltpu.sync_copy(zeros_hbm.at[pl.ds(s * rows_per_tile, rows_per_tile)],
                    acc_sh.at[pl.ds(s * rows_per_tile, rows_per_tile)])
    for i in range(_CHUNK // _LANES):
        ones_v[pl.ds(i * _LANES, _LANES)] = jnp.ones((_LANES,), jnp.float32)
    pltpu.sync_copy(dst_hbm.at[w], dst_v)
    plsc.subcore_barrier()

    def body(j, carry):
        pltpu.sync_copy(ones_v, acc_sh.at[dst_v.at[j]], add=True)
        return carry

    lax.fori_loop(0, nch, body, 0, unroll=False)
    plsc.subcore_barrier()
    pltpu.sync_copy(acc_sh.at[pl.ds(s * rows_per_tile, rows_per_tile)],
                    out_hbm.at[c, pl.ds(s * rows_per_tile, rows_per_tile)])


def _dis_from_parts(dp_hbm, row0, rows_per_tile, dis_v, inv_v, want_inv):
    pltpu.sync_copy(dp_hbm.at[0, pl.ds(row0, rows_per_tile)], dis_v)
    pltpu.sync_copy(dp_hbm.at[1, pl.ds(row0, rows_per_tile)], inv_v)
    for k in range(rows_per_tile // _LANES):
        sl = pl.ds(k * _LANES, _LANES)
        deg = dis_v[sl] + inv_v[sl] + 1.0
        dis_v[sl] = _rsqrt_sc(deg)
        if want_inv:
            inv_v[sl] = 1.0 / deg


def _stage_agg_inputs(c, s, src_hbm, dst_hbm, zeros_hbm,
                      src_v, dst_v, acc_sh, sem):
    w = s * 2 + c
    npad = zeros_hbm.shape[0]
    rows_per_tile = npad // 16
    return (
        pltpu.async_copy(
            zeros_hbm.at[pl.ds(s * rows_per_tile, rows_per_tile)],
            acc_sh.at[pl.ds(s * rows_per_tile, rows_per_tile)], sem),
        pltpu.async_copy(src_hbm.at[w], src_v, sem),
        pltpu.async_copy(dst_hbm.at[w], dst_v, sem),
    )


def _agg_phase(ytab_hbm, c, src_hbm, dst_hbm, zeros_hbm, out_hbm,
               src_v, dst_v, rows_v, sem, acc_sh, s):
    nch = src_hbm.shape[1]
    npad = zeros_hbm.shape[0]
    rows_per_tile = npad // 16

    plsc.subcore_barrier()

    def body(j, carry):
        pltpu.async_copy(ytab_hbm.at[c].at[src_v.at[j]], rows_v, sem).wait()
        pltpu.sync_copy(rows_v, acc_sh.at[dst_v.at[j]], add=True)
        return carry

    lax.fori_loop(0, nch, body, 0, unroll=False)
    plsc.subcore_barrier()
    pltpu.sync_copy(acc_sh.at[pl.ds(s * rows_per_tile, rows_per_tile)],
                    out_hbm.at[c, pl.ds(s * rows_per_tile, rows_per_tile)])


def _sc_agg1(xw_hbm, dp_hbm, src_hbm, dst_hbm, zeros_hbm,
             out_hbm, ytab_hbm,
             src_v, dst_v, rows_v, dis_v, inv_v, buf_v, sem, acc_sh):
    c = lax.axis_index("c")
    s = lax.axis_index("s")
    npad = dp_hbm.shape[1]
    rows_per_tile = npad // 16
    row0 = s * rows_per_tile

    staged = _stage_agg_inputs(c, s, src_hbm, dst_hbm, zeros_hbm,
                               src_v, dst_v, acc_sh, sem)
    _dis_from_parts(dp_hbm, row0, rows_per_tile, dis_v, inv_v, want_inv=False)
    pltpu.sync_copy(xw_hbm.at[pl.ds(row0, rows_per_tile)], buf_v)

    def ychunk(k, carry):
        dis16 = dis_v[pl.ds(k * _LANES, _LANES)]
        for i in range(_LANES):
            dsp = _splat(dis16, i)
            r = k * _LANES + i
            buf_v[r] = dsp * buf_v[r]
        return carry

    lax.fori_loop(0, rows_per_tile // _LANES, ychunk, 0, unroll=False)
    pltpu.sync_copy(buf_v, ytab_hbm.at[c, pl.ds(row0, rows_per_tile)])
    for cp in staged:
        cp.wait()

    _agg_phase(ytab_hbm, c, src_hbm, dst_hbm, zeros_hbm, out_hbm,
               src_v, dst_v, rows_v, sem, acc_sh, s)


def _sc_agg2(p_hbm, xw_hbm, dp_hbm, b1_hbm, src_hbm, dst_hbm, zeros_hbm,
             out_hbm, ytab_hbm,
             src_v, dst_v, rows_v, dis_v, inv_v, buf_v, p0_v, p1_v, b1_v,
             sem, acc_sh):
    c = lax.axis_index("c")
    s = lax.axis_index("s")
    npad = dp_hbm.shape[1]
    rows_per_tile = npad // 16
    row0 = s * rows_per_tile

    staged = _stage_agg_inputs(c, s, src_hbm, dst_hbm, zeros_hbm,
                               src_v, dst_v, acc_sh, sem)
    pltpu.sync_copy(b1_hbm, b1_v)
    _dis_from_parts(dp_hbm, row0, rows_per_tile, dis_v, inv_v, want_inv=False)
    pltpu.sync_copy(xw_hbm.at[pl.ds(row0, rows_per_tile)], buf_v)
    pltpu.sync_copy(p_hbm.at[0, pl.ds(row0, rows_per_tile)], p0_v)
    pltpu.sync_copy(p_hbm.at[1, pl.ds(row0, rows_per_tile)], p1_v)

    def ychunk(k, carry):
        dis16 = dis_v[pl.ds(k * _LANES, _LANES)]
        b1r = b1_v[...]
        for i in range(_LANES):
            dsp = _splat(dis16, i)
            isp = dsp * dsp
            r = k * _LANES + i
            pre = dsp * (p0_v[r] + p1_v[r]) + buf_v[r] * isp + b1r
            h = jnp.maximum(pre, 0.0)
            buf_v[r] = dsp * h
        return carry

    lax.fori_loop(0, rows_per_tile // _LANES, ychunk, 0, unroll=False)
    pltpu.sync_copy(buf_v, ytab_hbm.at[c, pl.ds(row0, rows_per_tile)])
    for cp in staged:
        cp.wait()

    _agg_phase(ytab_hbm, c, src_hbm, dst_hbm, zeros_hbm, out_hbm,
               src_v, dst_v, rows_v, sem, acc_sh, s)


def _run_sc_degree(dst_chunks, np_rows):
    mesh = plsc.VectorSubcoreMesh(core_axis_name="c", subcore_axis_name="s")
    nch = dst_chunks.shape[1]
    zeros = jnp.zeros((np_rows,), jnp.float32)
    kern = pl.kernel(
        _sc_degree,
        mesh=mesh,
        compiler_params=pltpu.CompilerParams(use_tc_tiling_on_sc=False),
        out_type=jax.ShapeDtypeStruct((2, np_rows), jnp.float32),
        scratch_types=[
            pltpu.VMEM((nch, _CHUNK), jnp.int32),
            pltpu.VMEM((_CHUNK,), jnp.float32),
            pltpu.VMEM_SHARED((np_rows,), jnp.float32),
        ],
    )
    return kern(dst_chunks, zeros)


def _run_sc_agg1(xw, dp, src_chunks, dst_chunks):
    mesh = plsc.VectorSubcoreMesh(core_axis_name="c", subcore_axis_name="s")
    nch = src_chunks.shape[1]
    np_rows = xw.shape[0]
    rpt = np_rows // 16
    zeros = jnp.zeros((np_rows, _LANES), jnp.float32)
    kern = pl.kernel(
        _sc_agg1,
        mesh=mesh,
        compiler_params=pltpu.CompilerParams(use_tc_tiling_on_sc=False),
        out_type=(
            jax.ShapeDtypeStruct((2, np_rows, _LANES), jnp.float32),
            jax.ShapeDtypeStruct((2, np_rows, _LANES), jnp.float32),
        ),
        scratch_types=[
            pltpu.VMEM((nch, _CHUNK), jnp.int32),
            pltpu.VMEM((nch, _CHUNK), jnp.int32),
            pltpu.VMEM((_CHUNK, _LANES), jnp.float32),
            pltpu.VMEM((rpt,), jnp.float32),
            pltpu.VMEM((rpt,), jnp.float32),
            pltpu.VMEM((rpt, _LANES), jnp.float32),
            pltpu.SemaphoreType.DMA,
            pltpu.VMEM_SHARED((np_rows, _LANES), jnp.float32),
        ],
    )
    p, _ = kern(xw, dp, src_chunks, dst_chunks, zeros)
    return p


def _run_sc_agg2(p, xw, dp, b1, src_chunks, dst_chunks):
    mesh = plsc.VectorSubcoreMesh(core_axis_name="c", subcore_axis_name="s")
    nch = src_chunks.shape[1]
    np_rows = xw.shape[0]
    rpt = np_rows // 16
    zeros = jnp.zeros((np_rows, _LANES), jnp.float32)
    kern = pl.kernel(
        _sc_agg2,
        mesh=mesh,
        compiler_params=pltpu.CompilerParams(use_tc_tiling_on_sc=False),
        out_type=(
            jax.ShapeDtypeStruct((2, np_rows, _LANES), jnp.float32),
            jax.ShapeDtypeStruct((2, np_rows, _LANES), jnp.float32),
        ),
        scratch_types=[
            pltpu.VMEM((nch, _CHUNK), jnp.int32),
            pltpu.VMEM((nch, _CHUNK), jnp.int32),
            pltpu.VMEM((_CHUNK, _LANES), jnp.float32),
            pltpu.VMEM((rpt,), jnp.float32),
            pltpu.VMEM((rpt,), jnp.float32),
            pltpu.VMEM((rpt, _LANES), jnp.float32),
            pltpu.VMEM((rpt, _LANES), jnp.float32),
            pltpu.VMEM((rpt, _LANES), jnp.float32),
            pltpu.VMEM((_LANES,), jnp.float32),
            pltpu.SemaphoreType.DMA,
            pltpu.VMEM_SHARED((np_rows, _LANES), jnp.float32),
        ],
    )
    q, _ = kern(p, xw, dp, b1, src_chunks, dst_chunks, zeros)
    return q


def _tc_dense1(x_ref, w1_ref, o_xw):
    o_xw[...] = jnp.dot(x_ref[...], w1_ref[...],
                        preferred_element_type=jnp.float32)


def _tc_dense3(q_ref, p_ref, xw_ref, dp_ref, b1_ref, w2_ref, b2_ref, o_out):
    n_real = o_out.shape[1]
    deg = dp_ref[0] + dp_ref[1] + 1.0
    deg = deg[:, None]
    dis = lax.rsqrt(deg)
    xw = xw_ref[...]
    h = jax.nn.relu(dis * (p_ref[0] + p_ref[1]) + xw / deg
                    + b1_ref[...][None, :])
    o = (q_ref[0] + q_ref[1]) * dis + h / deg
    logits_t = lax.dot_general(w2_ref[...], o, (((0,), (1,)), ((), ())),
                               preferred_element_type=jnp.float32)
    logits_t = logits_t + b2_ref[...][:, None]
    m = jnp.max(logits_t, axis=0, keepdims=True)
    lse = m + jnp.log(jnp.sum(jnp.exp(logits_t - m), axis=0, keepdims=True))
    o_out[...] = (logits_t - lse)[:, :n_real]


def kernel(x, edge_index, W1, b1, W2, b2):
    n, f_in = x.shape
    f_hid = W1.shape[1]
    f_out = W2.shape[1]
    e = edge_index.shape[1]

    np_rows = ((n // 256) + 1) * 256
    n_workers = 32
    nch = -(-e // (n_workers * _CHUNK))
    ep = n_workers * nch * _CHUNK
    n_spread = np_rows - n

    src = edge_index[0].astype(jnp.int32)
    dst = edge_index[1].astype(jnp.int32)
    pad_idx = n + jnp.arange(ep - e, dtype=jnp.int32) % n_spread
    src_chunks = jnp.concatenate([src, pad_idx]).reshape(n_workers, nch, _CHUNK)
    dst_chunks = jnp.concatenate([dst, pad_idx]).reshape(n_workers, nch, _CHUNK)

    x_pad = jnp.pad(x, ((0, np_rows - n), (0, 0)))

    dp = _run_sc_degree(dst_chunks, np_rows)
    xw = pl.pallas_call(
        _tc_dense1,
        out_shape=jax.ShapeDtypeStruct((np_rows, f_hid), jnp.float32),
    )(x_pad, W1)

    p = _run_sc_agg1(xw, dp, src_chunks, dst_chunks)

    q = _run_sc_agg2(p, xw, dp, b1, src_chunks, dst_chunks)

    out_t = pl.pallas_call(
        _tc_dense3,
        out_shape=jax.ShapeDtypeStruct((f_out, n), jnp.float32),
    )(q, p, xw, dp, b1, W2, b2)

    return out_t.T
